# Initial kernel scaffold; baseline (speedup 1.0000x reference)
import jax
import jax.numpy as jnp
from jax.experimental import pallas as pl


def kernel(scores, bbox_deltas, im_info, ignore_region, num_ignore):
    raise NotImplementedError("write your pallas kernel here")


# speedup vs baseline: 14.0405x; 14.0405x over previous
import numpy as np
import jax
import jax.numpy as jnp
from jax import lax
from jax.experimental import pallas as pl
from jax.experimental.pallas import tpu as pltpu

_FEAT_STRIDE = 16
_PRE = 6000
_POST = 300
_THRESH = 0.7
_FH, _FW, _A = 50, 75, 9
_N = _FH * _FW * _A
_ROWS = 264
_LANES = 128
_NPAD = _ROWS * _LANES
_B = 2

_INTERPRET = False


def _np_anchors():
    def whctrs(a):
        w = a[2] - a[0] + 1
        h = a[3] - a[1] + 1
        return w, h, a[0] + 0.5 * (w - 1), a[1] + 0.5 * (h - 1)

    def mk(ws, hs, xc, yc):
        ws = ws[:, None]
        hs = hs[:, None]
        return np.hstack((xc - 0.5 * (ws - 1), yc - 0.5 * (hs - 1),
                          xc + 0.5 * (ws - 1), yc + 0.5 * (hs - 1)))

    base = np.array([1, 1, _FEAT_STRIDE, _FEAT_STRIDE], dtype=np.float64) - 1
    ratios = np.array([0.5, 1.0, 2.0])
    scales = np.array([8, 16, 32])
    w, h, xc, yc = whctrs(base)
    size_ratios = (w * h) / ratios
    ws = np.round(np.sqrt(size_ratios))
    hs = np.round(ws * ratios)
    ra = mk(ws, hs, xc, yc)
    outs = []
    for i in range(ra.shape[0]):
        w, h, xc, yc = whctrs(ra[i, :])
        outs.append(mk(w * scales, h * scales, xc, yc))
    anch = np.vstack(outs).astype(np.float32)

    sx = np.arange(_FW, dtype=np.float32) * _FEAT_STRIDE
    sy = np.arange(_FH, dtype=np.float32) * _FEAT_STRIDE
    SX, SY = np.meshgrid(sx, sy)
    shifts = np.stack([SX.ravel(), SY.ravel(), SX.ravel(), SY.ravel()], 1)
    full = (anch[None, :, :] + shifts[:, None, :]).reshape(-1, 4)
    aw = full[:, 2] - full[:, 0] + 1.0
    ah = full[:, 3] - full[:, 1] + 1.0
    ax = full[:, 0] + 0.5 * aw
    ay = full[:, 1] + 0.5 * ah

    def pad(v, fill):
        return np.concatenate([v, np.full(_NPAD - _N, fill, np.float32)]
                              ).astype(np.float32).reshape(_ROWS, _LANES)

    return pad(aw, 1.0), pad(ah, 1.0), pad(ax, 0.0), pad(ay, 0.0)


_AW, _AH, _AX, _AY = _np_anchors()


def _i32c(v):
    v = int(v) & 0xFFFFFFFF
    if v >= 1 << 31:
        v -= 1 << 32
    return jnp.int32(v)


def _nms_body(info_ref, sc_ref, dx_ref, dy_ref, dw_ref, dh_ref,
              aw_ref, ah_ref, ax_ref, ay_ref, out_ref, vmask_ref):
    i32 = jnp.int32
    MIN32 = _i32c(0x80000000)
    lin = (lax.broadcasted_iota(i32, (_ROWS, _LANES), 0) * _LANES
           + lax.broadcasted_iota(i32, (_ROWS, _LANES), 1))
    in_range = lin < _N
    aw = aw_ref[...]
    ah = ah_ref[...]
    ax = ax_ref[...]
    ay = ay_ref[...]

    batches = []
    for b in range(_B):
        s = sc_ref[b]
        pcx = dx_ref[b] * aw + ax
        pcy = dy_ref[b] * ah + ay
        pw = jnp.exp(dw_ref[b]) * aw
        ph = jnp.exp(dh_ref[b]) * ah
        x1 = pcx - 0.5 * pw
        y1 = pcy - 0.5 * ph
        x2 = pcx + 0.5 * pw
        y2 = pcy + 0.5 * ph
        hm1 = info_ref[b, 0:1, :] - 1.0
        wm1 = info_ref[b, 1:2, :] - 1.0
        x1 = jnp.minimum(jnp.maximum(x1, 0.0), wm1)
        y1 = jnp.minimum(jnp.maximum(y1, 0.0), hm1)
        x2 = jnp.minimum(jnp.maximum(x2, 0.0), wm1)
        y2 = jnp.minimum(jnp.maximum(y2, 0.0), hm1)
        area = (x2 - x1 + 1.0) * (y2 - y1 + 1.0)

        bits = lax.bitcast_convert_type(s, i32)
        ku = jnp.where(bits >= 0, bits | MIN32, ~bits)
        ks = ku ^ MIN32

        p = i32(0)
        rem = i32(_PRE)
        for j in range(31, -1, -1):
            hi_mask = _i32c(~((1 << (j + 1)) - 1)) if j < 31 else i32(0)
            bit = _i32c(1 << j)
            match = ((ku & hi_mask) == p) & in_range
            cnt = jnp.sum(jnp.where(match & ((ku & bit) != 0), 1, 0))
            take = cnt >= rem
            p = jnp.where(take, p | bit, p)
            rem = jnp.where(take, rem, rem - cnt)
        tau_s = p ^ MIN32
        gt = (ks > tau_s) & in_range
        tie = (ks == tau_s) & in_range
        lo = i32(0)
        hi = i32(_N)
        for _ in range(16):
            mid = (lo + hi) // 2
            c = jnp.sum(jnp.where(tie & (lin < mid), 1, 0))
            geq = c >= rem
            hi = jnp.where(geq, mid, hi)
            lo = jnp.where(geq, lo, mid + 1)
        valid = gt | (tie & (lin < hi))
        vmask_ref[b] = jnp.where(valid, 1, 0)
        batches.append((ks, x1, y1, x2, y2, area))

    lane = lax.broadcasted_iota(i32, (1, _LANES), 1)
    IMAX = _i32c(0x7FFFFFFF)

    def step(k, carry):
        for b in range(_B):
            v = vmask_ref[b] != 0
            ks, x1, y1, x2, y2, area = batches[b]
            flag = jnp.any(v)
            mk = jnp.max(jnp.where(v, ks, MIN32))
            ii = jnp.min(jnp.where(v & (ks == mk), lin, IMAX))
            oh = lin == ii
            px1 = jnp.sum(jnp.where(oh, x1, 0.0))
            py1 = jnp.sum(jnp.where(oh, y1, 0.0))
            px2 = jnp.sum(jnp.where(oh, x2, 0.0))
            py2 = jnp.sum(jnp.where(oh, y2, 0.0))
            pa = jnp.sum(jnp.where(oh, area, 0.0))
            xx1 = jnp.maximum(px1, x1)
            yy1 = jnp.maximum(py1, y1)
            xx2 = jnp.minimum(px2, x2)
            yy2 = jnp.minimum(py2, y2)
            inter = (jnp.maximum(xx2 - xx1 + 1.0, 0.0)
                     * jnp.maximum(yy2 - yy1 + 1.0, 0.0))
            iou = inter / (pa + area - inter)
            v = v & (iou <= _THRESH) & flag
            fgate = jnp.where(flag, 1.0, 0.0)
            row = (jnp.where(lane == 0, px1, 0.0)
                   + jnp.where(lane == 1, py1, 0.0)
                   + jnp.where(lane == 2, px2, 0.0)
                   + jnp.where(lane == 3, py2, 0.0)) * fgate
            out_ref[b, pl.ds(k, 1), :] = row
            vmask_ref[b] = jnp.where(v, 1, 0)
        return carry

    lax.fori_loop(0, _POST, step, 0)


def _pad_to_grid(x):
    pad = jnp.zeros((_B, _NPAD - _N), x.dtype)
    return jnp.concatenate([x, pad], axis=1).reshape(_B, _ROWS, _LANES)


def kernel(scores, bbox_deltas, im_info, ignore_region, num_ignore):
    del ignore_region
    B = scores.shape[0]
    obj = jnp.transpose(scores[:, _A:, :, :], (0, 2, 3, 1)).reshape(B, _N)
    obj = obj + jnp.asarray(num_ignore, dtype=obj.dtype)
    dl = jnp.transpose(bbox_deltas, (0, 2, 3, 1)).reshape(B, _N, 4)

    neg = jnp.full((_B, _NPAD - _N), -jnp.inf, jnp.float32)
    sc = jnp.concatenate([obj, neg], axis=1).reshape(_B, _ROWS, _LANES)
    dx = _pad_to_grid(dl[..., 0])
    dy = _pad_to_grid(dl[..., 1])
    dw = _pad_to_grid(dl[..., 2])
    dh = _pad_to_grid(dl[..., 3])

    info = jnp.stack([
        jnp.broadcast_to(im_info[:, 0][:, None], (_B, _LANES)),
        jnp.broadcast_to(im_info[:, 1][:, None], (_B, _LANES)),
    ], axis=1)
    info = jnp.concatenate(
        [info, jnp.zeros((_B, 6, _LANES), jnp.float32)], axis=1)

    aw = jnp.asarray(_AW)
    ah = jnp.asarray(_AH)
    ax = jnp.asarray(_AX)
    ay = jnp.asarray(_AY)

    out = pl.pallas_call(
        _nms_body,
        out_shape=jax.ShapeDtypeStruct((_B, 304, _LANES), jnp.float32),
        scratch_shapes=[pltpu.VMEM((_B, _ROWS, _LANES), jnp.int32)],
        interpret=_INTERPRET,
    )(info, sc, dx, dy, dw, dh, aw, ah, ax, ay)

    kept = out[:, :_POST, :4]
    col0 = jnp.broadcast_to(
        jnp.arange(B, dtype=jnp.float32)[:, None, None], (B, _POST, 1))
    return jnp.concatenate([col0, kept], axis=2)


# speedup vs baseline: 15.3105x; 1.0905x over previous
import numpy as np
import jax
import jax.numpy as jnp
from jax import lax
from jax.experimental import pallas as pl
from jax.experimental.pallas import tpu as pltpu

_FEAT_STRIDE = 16
_PRE = 6000
_POST = 300
_THRESH = 0.7
_FH, _FW, _A = 50, 75, 9
_N = _FH * _FW * _A
_ROWS = 264
_LANES = 128
_NPAD = _ROWS * _LANES
_B = 2

_INTERPRET = False


def _np_anchors():
    def whctrs(a):
        w = a[2] - a[0] + 1
        h = a[3] - a[1] + 1
        return w, h, a[0] + 0.5 * (w - 1), a[1] + 0.5 * (h - 1)

    def mk(ws, hs, xc, yc):
        ws = ws[:, None]
        hs = hs[:, None]
        return np.hstack((xc - 0.5 * (ws - 1), yc - 0.5 * (hs - 1),
                          xc + 0.5 * (ws - 1), yc + 0.5 * (hs - 1)))

    base = np.array([1, 1, _FEAT_STRIDE, _FEAT_STRIDE], dtype=np.float64) - 1
    ratios = np.array([0.5, 1.0, 2.0])
    scales = np.array([8, 16, 32])
    w, h, xc, yc = whctrs(base)
    size_ratios = (w * h) / ratios
    ws = np.round(np.sqrt(size_ratios))
    hs = np.round(ws * ratios)
    ra = mk(ws, hs, xc, yc)
    outs = []
    for i in range(ra.shape[0]):
        w, h, xc, yc = whctrs(ra[i, :])
        outs.append(mk(w * scales, h * scales, xc, yc))
    anch = np.vstack(outs).astype(np.float32)

    sx = np.arange(_FW, dtype=np.float32) * _FEAT_STRIDE
    sy = np.arange(_FH, dtype=np.float32) * _FEAT_STRIDE
    SX, SY = np.meshgrid(sx, sy)
    shifts = np.stack([SX.ravel(), SY.ravel(), SX.ravel(), SY.ravel()], 1)
    full = (anch[None, :, :] + shifts[:, None, :]).reshape(-1, 4)
    aw = full[:, 2] - full[:, 0] + 1.0
    ah = full[:, 3] - full[:, 1] + 1.0
    ax = full[:, 0] + 0.5 * aw
    ay = full[:, 1] + 0.5 * ah

    def pad(v, fill):
        return np.concatenate([v, np.full(_NPAD - _N, fill, np.float32)]
                              ).astype(np.float32).reshape(_ROWS, _LANES)

    return pad(aw, 1.0), pad(ah, 1.0), pad(ax, 0.0), pad(ay, 0.0)


_AW, _AH, _AX, _AY = _np_anchors()


def _i32c(v):
    v = int(v) & 0xFFFFFFFF
    if v >= 1 << 31:
        v -= 1 << 32
    return jnp.int32(v)


def _nms_body(info_ref, sc_ref, dx_ref, dy_ref, dw_ref, dh_ref,
              aw_ref, ah_ref, ax_ref, ay_ref, out_ref, vmask_ref, plane_ref):
    i32 = jnp.int32
    MIN32 = _i32c(0x80000000)
    lin = (lax.broadcasted_iota(i32, (_ROWS, _LANES), 0) * _LANES
           + lax.broadcasted_iota(i32, (_ROWS, _LANES), 1))
    in_range = lin < _N
    aw = aw_ref[...]
    ah = ah_ref[...]
    ax = ax_ref[...]
    ay = ay_ref[...]

    batches = []
    for b in range(_B):
        s = sc_ref[b]
        pcx = dx_ref[b] * aw + ax
        pcy = dy_ref[b] * ah + ay
        pw = jnp.exp(dw_ref[b]) * aw
        ph = jnp.exp(dh_ref[b]) * ah
        x1 = pcx - 0.5 * pw
        y1 = pcy - 0.5 * ph
        x2 = pcx + 0.5 * pw
        y2 = pcy + 0.5 * ph
        hm1 = info_ref[b, 0:1, :] - 1.0
        wm1 = info_ref[b, 1:2, :] - 1.0
        x1 = jnp.minimum(jnp.maximum(x1, 0.0), wm1)
        y1 = jnp.minimum(jnp.maximum(y1, 0.0), hm1)
        x2 = jnp.minimum(jnp.maximum(x2, 0.0), wm1)
        y2 = jnp.minimum(jnp.maximum(y2, 0.0), hm1)
        area = (x2 - x1 + 1.0) * (y2 - y1 + 1.0)

        bits = lax.bitcast_convert_type(s, i32)
        ku = jnp.where(bits >= 0, bits | MIN32, ~bits)
        ks = ku ^ MIN32

        p = i32(0)
        rem = i32(_PRE)
        for j in range(31, -1, -1):
            hi_mask = _i32c(~((1 << (j + 1)) - 1)) if j < 31 else i32(0)
            bit = _i32c(1 << j)
            match = ((ku & hi_mask) == p) & in_range
            cnt = jnp.sum(jnp.where(match & ((ku & bit) != 0), 1, 0))
            take = cnt >= rem
            p = jnp.where(take, p | bit, p)
            rem = jnp.where(take, rem, rem - cnt)
        tau_s = p ^ MIN32
        gt = (ks > tau_s) & in_range
        tie = (ks == tau_s) & in_range
        lo = i32(0)
        hi = i32(_N)
        for _ in range(16):
            mid = (lo + hi) // 2
            c = jnp.sum(jnp.where(tie & (lin < mid), 1, 0))
            geq = c >= rem
            hi = jnp.where(geq, mid, hi)
            lo = jnp.where(geq, lo, mid + 1)
        valid = gt | (tie & (lin < hi))
        vmask_ref[b] = jnp.where(valid, 1, 0)
        plane_ref[b, 0] = x1
        plane_ref[b, 1] = y1
        plane_ref[b, 2] = x2
        plane_ref[b, 3] = y2
        plane_ref[b, 4] = area
        batches.append((ks, x1, y1, x2, y2, area))

    lane = lax.broadcasted_iota(i32, (1, _LANES), 1)
    IMAX = _i32c(0x7FFFFFFF)

    def step(k, carry):
        for b in range(_B):
            v = vmask_ref[b] != 0
            ks, x1, y1, x2, y2, area = batches[b]
            mk = jnp.max(jnp.where(v, ks, MIN32))
            flag = mk != MIN32
            ii = jnp.min(jnp.where(v & (ks == mk), lin, IMAX))
            iic = jnp.minimum(ii, i32(_N - 1))
            r0 = iic // _LANES
            l0 = iic - r0 * _LANES
            lsel = lane == l0
            px1 = jnp.sum(jnp.where(lsel, plane_ref[b, 0, pl.ds(r0, 1), :], 0.0))
            py1 = jnp.sum(jnp.where(lsel, plane_ref[b, 1, pl.ds(r0, 1), :], 0.0))
            px2 = jnp.sum(jnp.where(lsel, plane_ref[b, 2, pl.ds(r0, 1), :], 0.0))
            py2 = jnp.sum(jnp.where(lsel, plane_ref[b, 3, pl.ds(r0, 1), :], 0.0))
            pa = jnp.sum(jnp.where(lsel, plane_ref[b, 4, pl.ds(r0, 1), :], 0.0))
            xx1 = jnp.maximum(px1, x1)
            yy1 = jnp.maximum(py1, y1)
            xx2 = jnp.minimum(px2, x2)
            yy2 = jnp.minimum(py2, y2)
            inter = (jnp.maximum(xx2 - xx1 + 1.0, 0.0)
                     * jnp.maximum(yy2 - yy1 + 1.0, 0.0))
            iou = inter / (pa + area - inter)
            v = v & (iou <= _THRESH)
            fgate = jnp.where(flag, 1.0, 0.0)
            row = (jnp.where(lane == 0, px1, 0.0)
                   + jnp.where(lane == 1, py1, 0.0)
                   + jnp.where(lane == 2, px2, 0.0)
                   + jnp.where(lane == 3, py2, 0.0)) * fgate
            out_ref[b, pl.ds(k, 1), :] = row
            vmask_ref[b] = jnp.where(v, 1, 0)
        return carry

    lax.fori_loop(0, _POST, step, 0)


def _pad_to_grid(x):
    pad = jnp.zeros((_B, _NPAD - _N), x.dtype)
    return jnp.concatenate([x, pad], axis=1).reshape(_B, _ROWS, _LANES)


def kernel(scores, bbox_deltas, im_info, ignore_region, num_ignore):
    del ignore_region
    B = scores.shape[0]
    obj = jnp.transpose(scores[:, _A:, :, :], (0, 2, 3, 1)).reshape(B, _N)
    obj = obj + jnp.asarray(num_ignore, dtype=obj.dtype)
    dl = jnp.transpose(bbox_deltas, (0, 2, 3, 1)).reshape(B, _N, 4)

    neg = jnp.full((_B, _NPAD - _N), -jnp.inf, jnp.float32)
    sc = jnp.concatenate([obj, neg], axis=1).reshape(_B, _ROWS, _LANES)
    dx = _pad_to_grid(dl[..., 0])
    dy = _pad_to_grid(dl[..., 1])
    dw = _pad_to_grid(dl[..., 2])
    dh = _pad_to_grid(dl[..., 3])

    info = jnp.stack([
        jnp.broadcast_to(im_info[:, 0][:, None], (_B, _LANES)),
        jnp.broadcast_to(im_info[:, 1][:, None], (_B, _LANES)),
    ], axis=1)
    info = jnp.concatenate(
        [info, jnp.zeros((_B, 6, _LANES), jnp.float32)], axis=1)

    aw = jnp.asarray(_AW)
    ah = jnp.asarray(_AH)
    ax = jnp.asarray(_AX)
    ay = jnp.asarray(_AY)

    out = pl.pallas_call(
        _nms_body,
        out_shape=jax.ShapeDtypeStruct((_B, 304, _LANES), jnp.float32),
        scratch_shapes=[pltpu.VMEM((_B, _ROWS, _LANES), jnp.int32),
                        pltpu.VMEM((_B, 5, _ROWS, _LANES), jnp.float32)],
        interpret=_INTERPRET,
    )(info, sc, dx, dy, dw, dh, aw, ah, ax, ay)

    kept = out[:, :_POST, :4]
    col0 = jnp.broadcast_to(
        jnp.arange(B, dtype=jnp.float32)[:, None, None], (B, _POST, 1))
    return jnp.concatenate([col0, kept], axis=2)
